# baseline (speedup 1.0000x reference)
import functools

import jax
import jax.numpy as jnp
from jax import lax
from jax.experimental import pallas as pl
from jax.experimental.pallas import tpu as pltpu
from jax.experimental.pallas import tpu_sc as plsc

B, L = 16384, 200
N = B * L
V = 1_000_000
VP = 1 << 20
NC, NS = 2, 16
NW = NC * NS

CPW = N // NW
C = 2048
NCHUNK = CPW // C
SW = 256
G = C // SW

VPC = VP // NS
CV = 2048
NVCHUNK = VPC // CV


SB = 512


def _expand_body(w_ref, b_ref, tab_ref, out_ref):
    t = tab_ref[...]
    for k in range(3):
        out_ref[k] = t * w_ref[k] + b_ref[k]


def _expand_planes(tab2d, W3, b3):
    return pl.pallas_call(
        _expand_body,
        grid=(VP // 128 // SB,),
        in_specs=[
            pl.BlockSpec(memory_space=pltpu.SMEM),
            pl.BlockSpec(memory_space=pltpu.SMEM),
            pl.BlockSpec((SB, 128), lambda i: (i, 0)),
        ],
        out_specs=pl.BlockSpec((3, SB, 128), lambda i: (0, i, 0)),
        out_shape=jax.ShapeDtypeStruct((3, VP // 128, 128), jnp.float32),
    )(W3, b3, tab2d)


@functools.partial(
    pl.kernel,
    out_type=jax.ShapeDtypeStruct((N, 3), jnp.float32),
    mesh=plsc.VectorSubcoreMesh(core_axis_name="c", subcore_axis_name="s"),
    compiler_params=pltpu.CompilerParams(
        needs_layout_passes=False, use_tc_tiling_on_sc=False
    ),
    scratch_types=[
        pltpu.HBM((NC, VP, 8), jnp.float32),
        pltpu.VMEM((CV, 8), jnp.float32),
        pltpu.VMEM((2, C), jnp.int32),
        pltpu.VMEM((2, C, 8), jnp.float32),
        pltpu.SemaphoreType.DMA,
        pltpu.SemaphoreType.DMA,
        pltpu.SemaphoreType.DMA,
        pltpu.SemaphoreType.DMA,
        pltpu.SemaphoreType.DMA,
    ],
)
def _fused(planes_hbm, idx_hbm, out_hbm,
           t8d, buf8_v, idx_v, vals_v, sem_i0, sem_i1, sem_g, sem_o0, sem_o1):
    cid = lax.axis_index("c")
    sid = lax.axis_index("s")
    wid = sid * NC + cid

    vbase0 = sid * VPC

    @pl.loop(0, NVCHUNK)
    def vchunk(t):
        vbase = vbase0 + t * CV
        for k in range(3):
            pltpu.sync_copy(
                planes_hbm.at[pl.ds(k * VP + vbase, CV)],
                buf8_v.at[:, pl.ds(k, 1)],
            )
        pltpu.sync_copy(buf8_v, t8d.at[cid, pl.ds(vbase, CV)])

    plsc.subcore_barrier()

    t8 = t8d.at[cid]
    base0 = wid * CPW
    sem_i = (sem_i0, sem_i1)
    sem_o = (sem_o0, sem_o1)

    def idx_src(u):
        return idx_hbm.at[pl.ds(base0 + u * C, C)]

    def out_dst(u):
        return out_hbm.at[pl.ds(base0 + u * C, C)]

    def gathers(b):
        return [
            pltpu.async_copy(
                t8.at[idx_v.at[b, pl.ds(g * SW, SW)]],
                vals_v.at[b, pl.ds(g * SW, SW)],
                sem_g,
            )
            for g in range(G)
        ]

    pltpu.async_copy(idx_src(0), idx_v.at[0], sem_i0)
    pltpu.async_copy(idx_src(1), idx_v.at[1], sem_i1)

    for b in (0, 1):
        pltpu.make_async_copy(idx_src(b), idx_v.at[b], sem_i[b]).wait()
        for cp in gathers(b):
            cp.wait()
        pltpu.async_copy(idx_src(b + 2), idx_v.at[b], sem_i[b])
        pltpu.async_copy(
            vals_v.at[b, :, pl.ds(0, 3)], out_dst(b), sem_o[b]
        )

    @pl.loop(1, NCHUNK // 2 - 1)
    def pair(t):
        for b in (0, 1):
            u = 2 * t + b
            pltpu.make_async_copy(idx_src(u), idx_v.at[b], sem_i[b]).wait()
            pltpu.make_async_copy(
                vals_v.at[b, :, pl.ds(0, 3)], out_dst(u), sem_o[b]
            ).wait()
            for cp in gathers(b):
                cp.wait()

            @pl.when(u + 2 < NCHUNK)
            def _():
                pltpu.async_copy(idx_src(u + 2), idx_v.at[b], sem_i[b])

            pltpu.async_copy(
                vals_v.at[b, :, pl.ds(0, 3)], out_dst(u), sem_o[b]
            )

    for b in (0, 1):
        u = NCHUNK - 2 + b
        pltpu.make_async_copy(idx_src(u), idx_v.at[b], sem_i[b]).wait()
        pltpu.make_async_copy(
            vals_v.at[b, :, pl.ds(0, 3)], out_dst(u), sem_o[b]
        ).wait()
        for cp in gathers(b):
            cp.wait()
        pltpu.sync_copy(vals_v.at[b, :, pl.ds(0, 3)], out_dst(u))


def kernel(data, table, W, b):
    idx = data.reshape(-1)
    tab = jnp.pad(table.reshape(-1), (0, VP - V)).reshape(VP // 128, 128)
    planes = _expand_planes(tab, W.reshape(-1), b.reshape(-1))
    out = _fused(planes.reshape(-1, 1), idx)
    return out.reshape(B, L, 3)


# speedup vs baseline: 5.8330x; 5.8330x over previous
import functools

import jax
import jax.numpy as jnp
from jax import lax
from jax.experimental import pallas as pl
from jax.experimental.pallas import tpu as pltpu
from jax.experimental.pallas import tpu_sc as plsc

B, L = 16384, 200
N = B * L
V = 1_000_000
VP = 1 << 20
NC, NS = 2, 16
NW = NC * NS

CPW = N // NW
C = 2048
NCHUNK = CPW // C
SW = 256
G = C // SW
VSL = VP // NS


@functools.partial(
    pl.kernel,
    out_type=jax.ShapeDtypeStruct((3 * N,), jnp.float32),
    mesh=plsc.VectorSubcoreMesh(core_axis_name="c", subcore_axis_name="s"),
    compiler_params=pltpu.CompilerParams(
        needs_layout_passes=False, use_tc_tiling_on_sc=False
    ),
    scratch_types=[
        pltpu.VMEM_SHARED((VP,), jnp.float32),
        pltpu.VMEM((16,), jnp.float32),
        pltpu.VMEM((2, C), jnp.int32),
        pltpu.VMEM((2, C), jnp.float32),
        pltpu.VMEM((2, 3 * C), jnp.float32),
        pltpu.SemaphoreType.DMA,
        pltpu.SemaphoreType.DMA,
        pltpu.SemaphoreType.DMA,
        pltpu.SemaphoreType.DMA,
        pltpu.SemaphoreType.DMA,
    ],
)
def _fused(tab_hbm, wb_hbm, idx_hbm, out_hbm,
           tab_sp, wb_v, idx_v, sv_v, out3_v,
           sem_i0, sem_i1, sem_g, sem_o0, sem_o1):
    cid = lax.axis_index("c")
    sid = lax.axis_index("s")
    wid = sid * NC + cid

    pltpu.sync_copy(
        tab_hbm.at[pl.ds(sid * VSL, VSL)], tab_sp.at[pl.ds(sid * VSL, VSL)]
    )
    pltpu.sync_copy(wb_hbm, wb_v)
    plsc.subcore_barrier()

    ii = lax.broadcasted_iota(jnp.int32, (16,), 0)
    npat, wpat, bpat = [], [], []
    for r in range(3):
        m = ii + 16 * r
        kk = m % 3
        npat.append(m // 3)
        wpat.append(plsc.load_gather(wb_v, [kk]))
        bpat.append(plsc.load_gather(wb_v, [kk + 3]))

    base0 = wid * CPW
    sem_i = (sem_i0, sem_i1)
    sem_o = (sem_o0, sem_o1)

    def idx_src(u):
        return idx_hbm.at[pl.ds(base0 + u * C, C)]

    def out_dst(u):
        return out_hbm.at[pl.ds(3 * (base0 + u * C), 3 * C)]

    def gathers(b):
        return [
            pltpu.async_copy(
                tab_sp.at[idx_v.at[b, pl.ds(g * SW, SW)]],
                sv_v.at[b, pl.ds(g * SW, SW)],
                sem_g,
            )
            for g in range(G)
        ]

    def expand(b):
        @pl.loop(0, C // 16)
        def jloop(j):
            nb = j * 16
            for r in range(3):
                v = plsc.load_gather(sv_v.at[b], [npat[r] + nb])
                out3_v[b, pl.ds(j * 48 + r * 16, 16)] = v * wpat[r] + bpat[r]

    pltpu.async_copy(idx_src(0), idx_v.at[0], sem_i0)
    pltpu.async_copy(idx_src(1), idx_v.at[1], sem_i1)

    for b in (0, 1):
        pltpu.make_async_copy(idx_src(b), idx_v.at[b], sem_i[b]).wait()
        for cp in gathers(b):
            cp.wait()
        pltpu.async_copy(idx_src(b + 2), idx_v.at[b], sem_i[b])
        expand(b)
        pltpu.async_copy(out3_v.at[b], out_dst(b), sem_o[b])

    @pl.loop(1, NCHUNK // 2 - 1)
    def pair(t):
        for b in (0, 1):
            u = 2 * t + b
            pltpu.make_async_copy(idx_src(u), idx_v.at[b], sem_i[b]).wait()
            for cp in gathers(b):
                cp.wait()

            @pl.when(u + 2 < NCHUNK)
            def _():
                pltpu.async_copy(idx_src(u + 2), idx_v.at[b], sem_i[b])

            pltpu.make_async_copy(out3_v.at[b], out_dst(u), sem_o[b]).wait()
            expand(b)
            pltpu.async_copy(out3_v.at[b], out_dst(u), sem_o[b])

    for b in (0, 1):
        u = NCHUNK - 2 + b
        pltpu.make_async_copy(idx_src(u), idx_v.at[b], sem_i[b]).wait()
        for cp in gathers(b):
            cp.wait()
        pltpu.make_async_copy(out3_v.at[b], out_dst(u), sem_o[b]).wait()
        expand(b)
        pltpu.sync_copy(out3_v.at[b], out_dst(u))


def kernel(data, table, W, b):
    idx = data.reshape(-1)
    tab = jnp.pad(table.reshape(-1), (0, VP - V))
    wb = jnp.concatenate(
        [W.reshape(-1), b.reshape(-1), jnp.zeros((10,), jnp.float32)]
    )
    out = _fused(tab, wb, idx)
    return out.reshape(B, L, 3)
